# baseline (speedup 1.0000x reference)
import jax
import jax.numpy as jnp
from jax.experimental import pallas as pl
from jax.experimental.pallas import tpu as pltpu

H, KVH, DH = 16, 8, 128
EPS = 1e-6
NEG = -1e30
LOG2E = 1.4426950408889634

BM_PROJ = 256
BM_Q = 512


def _rms_norm_heads(t, w, extra_scale):
    rows = t.shape[0]
    n = t.shape[1] // DH
    t3 = t.reshape(rows, n, DH)
    var = jnp.mean(t3 * t3, axis=-1, keepdims=True)
    t3 = t3 * (jax.lax.rsqrt(var + EPS) * extra_scale)
    return (t3 * w.reshape(1, 1, DH)).reshape(rows, n * DH)


def _qkv_proj_kernel(x_ref, wq_ref, wk_ref, wv_ref, qw_ref, kw_ref,
                     q_ref, k_ref, v_ref):
    xb = x_ref[...].astype(jnp.bfloat16)
    dims = (((1,), (1,)), ((), ()))
    q = jax.lax.dot_general(xb, wq_ref[...], dims,
                            preferred_element_type=jnp.float32)
    k = jax.lax.dot_general(xb, wk_ref[...], dims,
                            preferred_element_type=jnp.float32)
    v = jax.lax.dot_general(xb, wv_ref[...], dims,
                            preferred_element_type=jnp.float32)
    qn = _rms_norm_heads(q, qw_ref[...], DH ** -0.5 * LOG2E)
    kn = _rms_norm_heads(k, kw_ref[...], 1.0)
    q_ref[...] = qn.astype(jnp.bfloat16)
    k_ref[...] = kn.astype(jnp.bfloat16)
    v_ref[...] = v.astype(jnp.bfloat16)


def _attn_kernel(q_ref, k_ref, v_ref, o_ref):
    i = pl.program_id(1)
    dims_nt = (((1,), (1,)), ((), ()))
    dims_nn = (((1,), (0,)), ((), ()))
    q = q_ref[...]

    def qk(j):
        kc = k_ref[pl.ds(j * BM_Q, BM_Q), :]
        return jax.lax.dot_general(q, kc, dims_nt,
                                   preferred_element_type=jnp.float32)

    def chunk(j, carry):
        acc, l, s = carry
        s_next = qk(j + 1)
        p = jnp.exp2(s)
        l = l + jnp.sum(p, axis=-1, keepdims=True)
        vc = v_ref[pl.ds(j * BM_Q, BM_Q), :]
        acc = acc + jax.lax.dot_general(p.astype(jnp.bfloat16), vc, dims_nn,
                                        preferred_element_type=jnp.float32)
        return acc, l, s_next

    acc = jnp.zeros((BM_Q, DH), jnp.float32)
    l = jnp.zeros((BM_Q, 1), jnp.float32)
    acc, l, s = jax.lax.fori_loop(0, i, chunk, (acc, l, qk(0)))

    row = jax.lax.broadcasted_iota(jnp.int32, (BM_Q, BM_Q), 0)
    col = jax.lax.broadcasted_iota(jnp.int32, (BM_Q, BM_Q), 1)
    p = jnp.exp2(jnp.where(row >= col, s, NEG))
    l = l + jnp.sum(p, axis=-1, keepdims=True)
    vc = v_ref[pl.ds(i * BM_Q, BM_Q), :]
    acc = acc + jax.lax.dot_general(p.astype(jnp.bfloat16), vc, dims_nn,
                                    preferred_element_type=jnp.float32)
    o_ref[...] = (acc / l).astype(jnp.bfloat16)


def _out_proj_kernel(a_ref, wo_ref, o_ref):
    o_ref[...] = jax.lax.dot_general(
        a_ref[...], wo_ref[...], (((1,), (1,)), ((), ())),
        preferred_element_type=jnp.float32)


def kernel(x, Wq, Wk, Wv, Wo, q_norm_w, k_norm_w):
    b, s, d = x.shape
    x2 = x.reshape(s, d)
    wq = Wq.astype(jnp.bfloat16)
    wk = Wk.astype(jnp.bfloat16)
    wv = Wv.astype(jnp.bfloat16)
    wo = Wo.astype(jnp.bfloat16)
    qw = q_norm_w.reshape(1, DH)
    kw = k_norm_w.reshape(1, DH)

    n_row_blocks = s // BM_PROJ
    q, k, v = pl.pallas_call(
        _qkv_proj_kernel,
        grid=(n_row_blocks,),
        in_specs=[
            pl.BlockSpec((BM_PROJ, d), lambda i: (i, 0)),
            pl.BlockSpec((H * DH, d), lambda i: (0, 0)),
            pl.BlockSpec((KVH * DH, d), lambda i: (0, 0)),
            pl.BlockSpec((KVH * DH, d), lambda i: (0, 0)),
            pl.BlockSpec((1, DH), lambda i: (0, 0)),
            pl.BlockSpec((1, DH), lambda i: (0, 0)),
        ],
        out_specs=[
            pl.BlockSpec((BM_PROJ, H * DH), lambda i: (i, 0)),
            pl.BlockSpec((BM_PROJ, KVH * DH), lambda i: (i, 0)),
            pl.BlockSpec((BM_PROJ, KVH * DH), lambda i: (i, 0)),
        ],
        out_shape=[
            jax.ShapeDtypeStruct((s, H * DH), jnp.bfloat16),
            jax.ShapeDtypeStruct((s, KVH * DH), jnp.bfloat16),
            jax.ShapeDtypeStruct((s, KVH * DH), jnp.bfloat16),
        ],
    )(x2, wq, wk, wv, qw, kw)

    n_q_blocks = s // BM_Q
    groups = H // KVH
    attn = pl.pallas_call(
        _attn_kernel,
        grid=(H, n_q_blocks),
        in_specs=[
            pl.BlockSpec((BM_Q, DH), lambda h, i: (i, h)),
            pl.BlockSpec((s, DH), lambda h, i: (0, h // groups)),
            pl.BlockSpec((s, DH), lambda h, i: (0, h // groups)),
        ],
        out_specs=pl.BlockSpec((BM_Q, DH), lambda h, i: (i, h)),
        out_shape=jax.ShapeDtypeStruct((s, H * DH), jnp.bfloat16),
    )(q, k, v)

    out = pl.pallas_call(
        _out_proj_kernel,
        grid=(n_row_blocks,),
        in_specs=[
            pl.BlockSpec((BM_PROJ, H * DH), lambda i: (i, 0)),
            pl.BlockSpec((d, H * DH), lambda i: (0, 0)),
        ],
        out_specs=pl.BlockSpec((BM_PROJ, d), lambda i: (i, 0)),
        out_shape=jax.ShapeDtypeStruct((s, d), jnp.float32),
    )(attn, wo)

    return out.reshape(b, s, d)


# speedup vs baseline: 1.1703x; 1.1703x over previous
import jax
import jax.numpy as jnp
from jax.experimental import pallas as pl
from jax.experimental.pallas import tpu as pltpu

H, KVH, DH = 16, 8, 128
EPS = 1e-6
NEG = -1e30
LOG2E = 1.4426950408889634

BM_PROJ = 256
BM_Q = 512


def _rms_norm_heads(t, w, extra_scale):
    rows = t.shape[0]
    n = t.shape[1] // DH
    t3 = t.reshape(rows, n, DH)
    var = jnp.mean(t3 * t3, axis=-1, keepdims=True)
    t3 = t3 * (jax.lax.rsqrt(var + EPS) * extra_scale)
    return (t3 * w.reshape(1, 1, DH)).reshape(rows, n * DH)


def _qkv_proj_kernel(x_ref, wq_ref, wk_ref, wv_ref, qw_ref, kw_ref,
                     q_ref, k_ref, v_ref):
    xb = x_ref[...].astype(jnp.bfloat16)
    dims = (((1,), (1,)), ((), ()))
    q = jax.lax.dot_general(xb, wq_ref[...], dims,
                            preferred_element_type=jnp.float32)
    k = jax.lax.dot_general(xb, wk_ref[...], dims,
                            preferred_element_type=jnp.float32)
    v = jax.lax.dot_general(xb, wv_ref[...], dims,
                            preferred_element_type=jnp.float32)
    qn = _rms_norm_heads(q, qw_ref[...], DH ** -0.5 * LOG2E)
    kn = _rms_norm_heads(k, kw_ref[...], 1.0)
    q_ref[...] = qn.astype(jnp.bfloat16)
    k_ref[...] = kn.astype(jnp.bfloat16)
    v_ref[...] = v.astype(jnp.bfloat16)


def _attn_kernel(q_ref, k_ref, v_ref, o_ref):
    i = pl.program_id(1)
    dims_nt = (((1,), (1,)), ((), ()))
    dims_nn = (((1,), (0,)), ((), ()))
    q = q_ref[...]

    def chunk(j, carry):
        acc, l = carry
        kc = k_ref[pl.ds(j * BM_Q, BM_Q), :]
        s = jax.lax.dot_general(q, kc, dims_nt,
                                preferred_element_type=jnp.float32)
        p = jnp.exp2(s)
        l = l + jnp.sum(p, axis=-1, keepdims=True)
        vc = v_ref[pl.ds(j * BM_Q, BM_Q), :]
        acc = acc + jax.lax.dot_general(p.astype(jnp.bfloat16), vc, dims_nn,
                                        preferred_element_type=jnp.float32)
        return acc, l

    def chunk2(jj, carry):
        acc, l = carry
        j0 = jj * 2
        kcA = k_ref[pl.ds(j0 * BM_Q, BM_Q), :]
        sA = jax.lax.dot_general(q, kcA, dims_nt,
                                 preferred_element_type=jnp.float32)
        kcB = k_ref[pl.ds((j0 + 1) * BM_Q, BM_Q), :]
        sB = jax.lax.dot_general(q, kcB, dims_nt,
                                 preferred_element_type=jnp.float32)
        pA = jnp.exp2(sA)
        pB = jnp.exp2(sB)
        l = l + jnp.sum(pA, axis=-1, keepdims=True)
        l = l + jnp.sum(pB, axis=-1, keepdims=True)
        vcA = v_ref[pl.ds(j0 * BM_Q, BM_Q), :]
        acc = acc + jax.lax.dot_general(pA.astype(jnp.bfloat16), vcA, dims_nn,
                                        preferred_element_type=jnp.float32)
        vcB = v_ref[pl.ds((j0 + 1) * BM_Q, BM_Q), :]
        acc = acc + jax.lax.dot_general(pB.astype(jnp.bfloat16), vcB, dims_nn,
                                        preferred_element_type=jnp.float32)
        return acc, l

    acc = jnp.zeros((BM_Q, DH), jnp.float32)
    l = jnp.zeros((BM_Q, 1), jnp.float32)
    acc, l = jax.lax.fori_loop(0, i // 2, chunk2, (acc, l))
    acc, l = jax.lax.cond(i % 2 == 1,
                          lambda c: chunk(i - 1, c),
                          lambda c: c,
                          (acc, l))

    kc = k_ref[pl.ds(i * BM_Q, BM_Q), :]
    s = jax.lax.dot_general(q, kc, dims_nt,
                            preferred_element_type=jnp.float32)
    row = jax.lax.broadcasted_iota(jnp.int32, (BM_Q, BM_Q), 0)
    col = jax.lax.broadcasted_iota(jnp.int32, (BM_Q, BM_Q), 1)
    p = jnp.exp2(jnp.where(row >= col, s, NEG))
    l = l + jnp.sum(p, axis=-1, keepdims=True)
    vc = v_ref[pl.ds(i * BM_Q, BM_Q), :]
    acc = acc + jax.lax.dot_general(p.astype(jnp.bfloat16), vc, dims_nn,
                                    preferred_element_type=jnp.float32)
    o_ref[...] = (acc / l).astype(jnp.bfloat16)


def _out_proj_kernel(a_ref, wo_ref, o_ref):
    o_ref[...] = jax.lax.dot_general(
        a_ref[...], wo_ref[...], (((1,), (1,)), ((), ())),
        preferred_element_type=jnp.float32)


def kernel(x, Wq, Wk, Wv, Wo, q_norm_w, k_norm_w):
    b, s, d = x.shape
    x2 = x.reshape(s, d)
    wq = Wq.astype(jnp.bfloat16)
    wk = Wk.astype(jnp.bfloat16)
    wv = Wv.astype(jnp.bfloat16)
    wo = Wo.astype(jnp.bfloat16)
    qw = q_norm_w.reshape(1, DH)
    kw = k_norm_w.reshape(1, DH)

    n_row_blocks = s // BM_PROJ
    q, k, v = pl.pallas_call(
        _qkv_proj_kernel,
        grid=(n_row_blocks,),
        in_specs=[
            pl.BlockSpec((BM_PROJ, d), lambda i: (i, 0)),
            pl.BlockSpec((H * DH, d), lambda i: (0, 0)),
            pl.BlockSpec((KVH * DH, d), lambda i: (0, 0)),
            pl.BlockSpec((KVH * DH, d), lambda i: (0, 0)),
            pl.BlockSpec((1, DH), lambda i: (0, 0)),
            pl.BlockSpec((1, DH), lambda i: (0, 0)),
        ],
        out_specs=[
            pl.BlockSpec((BM_PROJ, H * DH), lambda i: (i, 0)),
            pl.BlockSpec((BM_PROJ, KVH * DH), lambda i: (i, 0)),
            pl.BlockSpec((BM_PROJ, KVH * DH), lambda i: (i, 0)),
        ],
        out_shape=[
            jax.ShapeDtypeStruct((s, H * DH), jnp.bfloat16),
            jax.ShapeDtypeStruct((s, KVH * DH), jnp.bfloat16),
            jax.ShapeDtypeStruct((s, KVH * DH), jnp.bfloat16),
        ],
    )(x2, wq, wk, wv, qw, kw)

    n_q_blocks = s // BM_Q
    groups = H // KVH
    attn = pl.pallas_call(
        _attn_kernel,
        grid=(H, n_q_blocks),
        in_specs=[
            pl.BlockSpec((BM_Q, DH), lambda h, i: (i, h)),
            pl.BlockSpec((s, DH), lambda h, i: (0, h // groups)),
            pl.BlockSpec((s, DH), lambda h, i: (0, h // groups)),
        ],
        out_specs=pl.BlockSpec((BM_Q, DH), lambda h, i: (i, h)),
        out_shape=jax.ShapeDtypeStruct((s, H * DH), jnp.bfloat16),
    )(q, k, v)

    out = pl.pallas_call(
        _out_proj_kernel,
        grid=(n_row_blocks,),
        in_specs=[
            pl.BlockSpec((BM_PROJ, H * DH), lambda i: (i, 0)),
            pl.BlockSpec((d, H * DH), lambda i: (0, 0)),
        ],
        out_specs=pl.BlockSpec((BM_PROJ, d), lambda i: (i, 0)),
        out_shape=jax.ShapeDtypeStruct((s, d), jnp.float32),
    )(attn, wo)

    return out.reshape(b, s, d)


# speedup vs baseline: 1.5089x; 1.2893x over previous
import jax
import jax.numpy as jnp
from jax.experimental import pallas as pl
from jax.experimental.pallas import tpu as pltpu

H, KVH, DH = 16, 8, 128
EPS = 1e-6
NEG = -1e30
LOG2E = 1.4426950408889634

BM_PROJ = 256
BM_Q = 1024
GQ = H // KVH


def _rms_norm_heads(t, w, extra_scale):
    rows = t.shape[0]
    n = t.shape[1] // DH
    t3 = t.reshape(rows, n, DH)
    var = jnp.mean(t3 * t3, axis=-1, keepdims=True)
    t3 = t3 * (jax.lax.rsqrt(var + EPS) * extra_scale)
    return (t3 * w.reshape(1, 1, DH)).reshape(rows, n * DH)


def _qkv_proj_kernel(x_ref, wq_ref, wk_ref, wv_ref, qw_ref, kw_ref,
                     q_ref, k_ref, v_ref, wqb_ref, wkb_ref, wvb_ref):
    @pl.when(pl.program_id(0) == 0)
    def _cast_weights():
        wqb_ref[...] = wq_ref[...].astype(jnp.bfloat16)
        wkb_ref[...] = wk_ref[...].astype(jnp.bfloat16)
        wvb_ref[...] = wv_ref[...].astype(jnp.bfloat16)

    xb = x_ref[...].astype(jnp.bfloat16)
    dims = (((1,), (1,)), ((), ()))
    q = jax.lax.dot_general(xb, wqb_ref[...], dims,
                            preferred_element_type=jnp.float32)
    k = jax.lax.dot_general(xb, wkb_ref[...], dims,
                            preferred_element_type=jnp.float32)
    v = jax.lax.dot_general(xb, wvb_ref[...], dims,
                            preferred_element_type=jnp.float32)
    qn = _rms_norm_heads(q, qw_ref[...], DH ** -0.5 * LOG2E)
    kn = _rms_norm_heads(k, kw_ref[...], 1.0)
    q_ref[...] = qn.astype(jnp.bfloat16)
    k_ref[...] = kn.astype(jnp.bfloat16)
    v_ref[...] = v.astype(jnp.bfloat16)


def _attn_kernel(q_ref, k_ref, v_ref, o_ref):
    i = pl.program_id(1)
    dims_nt = (((1,), (1,)), ((), ()))
    dims_nn = (((1,), (0,)), ((), ()))
    qA = q_ref[:, :DH]
    qB = q_ref[:, DH:]

    def chunk(j, carry, mask):
        accA, lA, accB, lB = carry
        kc = k_ref[pl.ds(j * BM_Q, BM_Q), :]
        sA = jax.lax.dot_general(qA, kc, dims_nt,
                                 preferred_element_type=jnp.float32)
        sB = jax.lax.dot_general(qB, kc, dims_nt,
                                 preferred_element_type=jnp.float32)
        if mask:
            row = jax.lax.broadcasted_iota(jnp.int32, (BM_Q, BM_Q), 0)
            col = jax.lax.broadcasted_iota(jnp.int32, (BM_Q, BM_Q), 1)
            keep = row >= col
            sA = jnp.where(keep, sA, NEG)
            sB = jnp.where(keep, sB, NEG)
        pA = jnp.exp2(sA)
        pB = jnp.exp2(sB)
        lA = lA + jnp.sum(pA, axis=-1, keepdims=True)
        lB = lB + jnp.sum(pB, axis=-1, keepdims=True)
        vc = v_ref[pl.ds(j * BM_Q, BM_Q), :]
        accA = accA + jax.lax.dot_general(pA.astype(jnp.bfloat16), vc,
                                          dims_nn,
                                          preferred_element_type=jnp.float32)
        accB = accB + jax.lax.dot_general(pB.astype(jnp.bfloat16), vc,
                                          dims_nn,
                                          preferred_element_type=jnp.float32)
        return accA, lA, accB, lB

    zero = (jnp.zeros((BM_Q, DH), jnp.float32),
            jnp.zeros((BM_Q, 1), jnp.float32),
            jnp.zeros((BM_Q, DH), jnp.float32),
            jnp.zeros((BM_Q, 1), jnp.float32))
    accA, lA, accB, lB = jax.lax.fori_loop(
        0, i, lambda j, c: chunk(j, c, False), zero)
    accA, lA, accB, lB = chunk(i, (accA, lA, accB, lB), True)
    o_ref[:, :DH] = (accA / lA).astype(jnp.bfloat16)
    o_ref[:, DH:] = (accB / lB).astype(jnp.bfloat16)


def _out_proj_kernel(a_ref, wo_ref, o_ref, wob_ref):
    @pl.when(pl.program_id(0) == 0)
    def _cast_weights():
        wob_ref[...] = wo_ref[...].astype(jnp.bfloat16)

    o_ref[...] = jax.lax.dot_general(
        a_ref[...], wob_ref[...], (((1,), (1,)), ((), ())),
        preferred_element_type=jnp.float32)


def kernel(x, Wq, Wk, Wv, Wo, q_norm_w, k_norm_w):
    b, s, d = x.shape
    x2 = x.reshape(s, d)
    qw = q_norm_w.reshape(1, DH)
    kw = k_norm_w.reshape(1, DH)

    n_row_blocks = s // BM_PROJ
    q, k, v = pl.pallas_call(
        _qkv_proj_kernel,
        grid=(n_row_blocks,),
        in_specs=[
            pl.BlockSpec((BM_PROJ, d), lambda i: (i, 0)),
            pl.BlockSpec((H * DH, d), lambda i: (0, 0)),
            pl.BlockSpec((KVH * DH, d), lambda i: (0, 0)),
            pl.BlockSpec((KVH * DH, d), lambda i: (0, 0)),
            pl.BlockSpec((1, DH), lambda i: (0, 0)),
            pl.BlockSpec((1, DH), lambda i: (0, 0)),
        ],
        out_specs=[
            pl.BlockSpec((BM_PROJ, H * DH), lambda i: (i, 0)),
            pl.BlockSpec((BM_PROJ, KVH * DH), lambda i: (i, 0)),
            pl.BlockSpec((BM_PROJ, KVH * DH), lambda i: (i, 0)),
        ],
        out_shape=[
            jax.ShapeDtypeStruct((s, H * DH), jnp.bfloat16),
            jax.ShapeDtypeStruct((s, KVH * DH), jnp.bfloat16),
            jax.ShapeDtypeStruct((s, KVH * DH), jnp.bfloat16),
        ],
        scratch_shapes=[
            pltpu.VMEM((H * DH, d), jnp.bfloat16),
            pltpu.VMEM((KVH * DH, d), jnp.bfloat16),
            pltpu.VMEM((KVH * DH, d), jnp.bfloat16),
        ],
    )(x2, Wq, Wk, Wv, qw, kw)

    n_q_blocks = s // BM_Q
    attn = pl.pallas_call(
        _attn_kernel,
        grid=(KVH, n_q_blocks),
        in_specs=[
            pl.BlockSpec((BM_Q, GQ * DH), lambda g, i: (i, g)),
            pl.BlockSpec((s, DH), lambda g, i: (0, g)),
            pl.BlockSpec((s, DH), lambda g, i: (0, g)),
        ],
        out_specs=pl.BlockSpec((BM_Q, GQ * DH), lambda g, i: (i, g)),
        out_shape=jax.ShapeDtypeStruct((s, H * DH), jnp.bfloat16),
    )(q, k, v)

    out = pl.pallas_call(
        _out_proj_kernel,
        grid=(n_row_blocks,),
        in_specs=[
            pl.BlockSpec((BM_PROJ, H * DH), lambda i: (i, 0)),
            pl.BlockSpec((d, H * DH), lambda i: (0, 0)),
        ],
        out_specs=pl.BlockSpec((BM_PROJ, d), lambda i: (i, 0)),
        out_shape=jax.ShapeDtypeStruct((s, d), jnp.float32),
        scratch_shapes=[
            pltpu.VMEM((d, H * DH), jnp.bfloat16),
        ],
    )(attn, Wo)

    return out.reshape(b, s, d)
